# baseline (speedup 1.0000x reference)
import functools

import jax
import jax.numpy as jnp
from jax import lax
from jax.experimental import pallas as pl
from jax.experimental.pallas import tpu as pltpu
from jax.experimental.pallas import tpu_sc as plsc

_C = 1
_O = 64
_L = 50
_N = 100000
_F = 128

_BTC = 2048
_KTC = 26
_NTC = _KTC * _BTC
_NSC = _N - _NTC

_LOAD = 128
_NLOAD = _NSC // _LOAD
_TROWS = _NSC - _NLOAD * _LOAD
_NBUF = 4


def _seg_sum_sc(x, labels):
    info = plsc.get_sparse_core_info()
    nc, ns = info.num_cores, info.num_subcores
    nw = nc * ns

    nslots = (_NLOAD + nw - 1) // nw
    last_cut = _NLOAD - (nslots - 1) * nw

    mesh = plsc.VectorSubcoreMesh(core_axis_name="c", subcore_axis_name="s")

    @functools.partial(
        pl.kernel,
        out_type=jax.ShapeDtypeStruct((nc, _L, _F), jnp.float32),
        mesh=mesh,
        scratch_types=[
            pltpu.VMEM((_NBUF, _LOAD, _F), jnp.float32),
            pltpu.VMEM((_NBUF, _LOAD), jnp.int32),
            pltpu.VMEM((_TROWS, _F), jnp.float32),
            pltpu.VMEM((_TROWS,), jnp.int32),
            pltpu.VMEM((_L, _F), jnp.float32),
            pltpu.VMEM_SHARED((_L, _F), jnp.float32),
            pltpu.SemaphoreType.DMA,
            pltpu.SemaphoreType.DMA,
            pltpu.SemaphoreType.DMA,
            pltpu.SemaphoreType.DMA,
            pltpu.SemaphoreType.DMA,
            pltpu.SemaphoreType.DMA,
            pltpu.SemaphoreType.DMA,
            pltpu.SemaphoreType.DMA,
            pltpu.SemaphoreType.DMA,
            pltpu.SemaphoreType.DMA,
            pltpu.SemaphoreType.DMA,
            pltpu.SemaphoreType.DMA,
            pltpu.SemaphoreType.DMA,
        ],
    )
    def seg_kernel(x_hbm, lbl_hbm, out_hbm, x_v, lbl_v, xt_v, lblt_v, zero_v,
                   acc_sh, sx0, sx1, sx2, sx3, sl0, sl1, sl2, sl3,
                   sc0, sc1, sc2, sc3, sq):
        cid = lax.axis_index("c")
        sid = lax.axis_index("s")
        wid = sid * nc + cid
        sx = (sx0, sx1, sx2, sx3)
        sl = (sl0, sl1, sl2, sl3)
        sc = (sc0, sc1, sc2, sc3)

        def mk_loads(i, b):
            m = wid + i * nw
            row0 = pl.multiple_of(_NTC + m * _LOAD, _LOAD)
            dl = pltpu.make_async_copy(
                lbl_hbm.at[pl.ds(row0, _LOAD)], lbl_v.at[b], sl[b])
            dx = pltpu.make_async_copy(
                x_hbm.at[pl.ds(row0, _LOAD), :], x_v.at[b], sx[b])
            return dl, dx

        def start_loads(i, b):
            for d in mk_loads(i, b):
                d.start()

        def mk_scat(b):
            return pltpu.make_async_copy(
                x_v.at[b], acc_sh.at[lbl_v.at[b]], sc[b])

        start_loads(0, 0)
        start_loads(1, 1)
        t0 = _NTC + _NLOAD * _LOAD
        dtl = pltpu.make_async_copy(lbl_hbm.at[pl.ds(t0, _TROWS)], lblt_v, sq)
        dtx = pltpu.make_async_copy(x_hbm.at[pl.ds(t0, _TROWS), :], xt_v, sq)

        @pl.when(wid == nw - 1)
        def _():
            dtl.start()
            dtx.start()

        @pl.when(sid == 0)
        def _():
            for l in range(_L):
                for j in range(_F // 16):
                    zero_v[l, pl.ds(j * 16, 16)] = jnp.zeros((16,), jnp.float32)
            pltpu.sync_copy(zero_v, acc_sh)

        plsc.subcore_barrier()

        for i in range(nslots):
            b = i % _NBUF

            def body(i=i, b=b):
                if i >= 2:
                    mk_scat((i - 2) % _NBUF).wait()
                nxt = i + 2
                if nxt < nslots - 1:
                    start_loads(nxt, nxt % _NBUF)
                elif nxt == nslots - 1:
                    @pl.when(wid < last_cut)
                    def _():
                        start_loads(nxt, nxt % _NBUF)
                for d in mk_loads(i, b):
                    d.wait()
                mk_scat(b).start(add=True)

            if i < nslots - 1:
                body()
            else:
                pl.when(wid < last_cut)(body)

        @pl.when(wid < last_cut)
        def _():
            mk_scat((nslots - 2) % _NBUF).wait()
            mk_scat((nslots - 1) % _NBUF).wait()

        @pl.when(jnp.logical_not(wid < last_cut))
        def _():
            mk_scat((nslots - 3) % _NBUF).wait()
            mk_scat((nslots - 2) % _NBUF).wait()

        @pl.when(wid == nw - 1)
        def _():
            dtl.wait()
            dtx.wait()
            pltpu.sync_copy(xt_v, acc_sh.at[lblt_v], add=True)

        plsc.subcore_barrier()

        @pl.when(sid == 0)
        def _():
            pltpu.sync_copy(acc_sh, out_hbm.at[cid])

    return seg_kernel(x, labels)


def _seg_sum_tc(x, labels_tc):
    def body(lbl_ref, x_ref, o_ref):
        k = pl.program_id(0)

        @pl.when(k == 0)
        def _():
            o_ref[...] = jnp.zeros_like(o_ref)

        lbl = lbl_ref[...].reshape(1, _BTC)
        rows = lax.broadcasted_iota(jnp.int32, (_O, _BTC), 0)
        oh = (rows == jnp.broadcast_to(lbl, (_O, _BTC))).astype(jnp.bfloat16)
        xb = x_ref[...].astype(jnp.bfloat16)
        o_ref[...] += jnp.dot(oh, xb, preferred_element_type=jnp.float32)

    return pl.pallas_call(
        body,
        grid=(_KTC,),
        in_specs=[
            pl.BlockSpec((_BTC,), lambda k: (k,)),
            pl.BlockSpec((_BTC, _F), lambda k: (k, 0)),
        ],
        out_specs=pl.BlockSpec((_O, _F), lambda k: (0, 0)),
        out_shape=jax.ShapeDtypeStruct((_O, _F), jnp.float32),
    )(labels_tc, x)


def _combine_tc(partials, tc_part, w2, b):
    def tc_body(p_ref, t_ref, w_ref, b_ref, o_ref):
        seg = p_ref[0] + p_ref[1] + t_ref[pl.ds(0, _L), :]
        o_ref[...] = (
            jax.lax.dot(w_ref[...], seg, preferred_element_type=jnp.float32)
            + b_ref[0]
        )

    return pl.pallas_call(
        tc_body,
        out_shape=jax.ShapeDtypeStruct((_O, _F), jnp.float32),
    )(partials, tc_part, w2, b)


def kernel(x, node_labels, Param_W, Param_b):
    labels = node_labels.astype(jnp.int32)
    sc_part = _seg_sum_sc(x, labels)
    tc_part = _seg_sum_tc(x, labels)
    w2 = Param_W.reshape(_O, _L)
    out = _combine_tc(sc_part, tc_part, w2, Param_b)
    return out.reshape(_C, _O, _F)


# speedup vs baseline: 1.0547x; 1.0547x over previous
import functools

import jax
import jax.numpy as jnp
from jax import lax
from jax.experimental import pallas as pl
from jax.experimental.pallas import tpu as pltpu
from jax.experimental.pallas import tpu_sc as plsc

_C = 1
_O = 64
_L = 50
_N = 100000
_F = 128

_BTC = 4096
_KTC = 12
_NTC = _KTC * _BTC
_NSC = _N - _NTC

_LOAD = 128
_NLOAD = _NSC // _LOAD
_TROWS = _NSC - _NLOAD * _LOAD
_NBUF = 4


def _seg_sum_sc(x, labels):
    info = plsc.get_sparse_core_info()
    nc, ns = info.num_cores, info.num_subcores
    nw = nc * ns

    nslots = (_NLOAD + nw - 1) // nw
    last_cut = _NLOAD - (nslots - 1) * nw

    mesh = plsc.VectorSubcoreMesh(core_axis_name="c", subcore_axis_name="s")

    @functools.partial(
        pl.kernel,
        out_type=jax.ShapeDtypeStruct((nc, _L, _F), jnp.float32),
        mesh=mesh,
        scratch_types=[
            pltpu.VMEM((_NBUF, _LOAD, _F), jnp.float32),
            pltpu.VMEM((_NBUF, _LOAD), jnp.int32),
            pltpu.VMEM((_TROWS, _F), jnp.float32),
            pltpu.VMEM((_TROWS,), jnp.int32),
            pltpu.VMEM((_L, _F), jnp.float32),
            pltpu.VMEM_SHARED((_L, _F), jnp.float32),
            pltpu.SemaphoreType.DMA,
            pltpu.SemaphoreType.DMA,
            pltpu.SemaphoreType.DMA,
            pltpu.SemaphoreType.DMA,
            pltpu.SemaphoreType.DMA,
            pltpu.SemaphoreType.DMA,
            pltpu.SemaphoreType.DMA,
            pltpu.SemaphoreType.DMA,
            pltpu.SemaphoreType.DMA,
            pltpu.SemaphoreType.DMA,
            pltpu.SemaphoreType.DMA,
            pltpu.SemaphoreType.DMA,
            pltpu.SemaphoreType.DMA,
        ],
    )
    def seg_kernel(x_hbm, lbl_hbm, out_hbm, x_v, lbl_v, xt_v, lblt_v, zero_v,
                   acc_sh, sx0, sx1, sx2, sx3, sl0, sl1, sl2, sl3,
                   sc0, sc1, sc2, sc3, sq):
        cid = lax.axis_index("c")
        sid = lax.axis_index("s")
        wid = sid * nc + cid
        sx = (sx0, sx1, sx2, sx3)
        sl = (sl0, sl1, sl2, sl3)
        sc = (sc0, sc1, sc2, sc3)

        def mk_loads(i, b):
            m = wid + i * nw
            row0 = pl.multiple_of(_NTC + m * _LOAD, _LOAD)
            dl = pltpu.make_async_copy(
                lbl_hbm.at[pl.ds(row0, _LOAD)], lbl_v.at[b], sl[b])
            dx = pltpu.make_async_copy(
                x_hbm.at[pl.ds(row0, _LOAD), :], x_v.at[b], sx[b])
            return dl, dx

        def start_loads(i, b):
            for d in mk_loads(i, b):
                d.start()

        def mk_scat(b):
            return pltpu.make_async_copy(
                x_v.at[b], acc_sh.at[lbl_v.at[b]], sc[b])

        start_loads(0, 0)
        start_loads(1, 1)
        t0 = _NTC + _NLOAD * _LOAD
        dtl = pltpu.make_async_copy(lbl_hbm.at[pl.ds(t0, _TROWS)], lblt_v, sq)
        dtx = pltpu.make_async_copy(x_hbm.at[pl.ds(t0, _TROWS), :], xt_v, sq)

        @pl.when(wid == nw - 1)
        def _():
            dtl.start()
            dtx.start()

        @pl.when(sid == 0)
        def _():
            for l in range(_L):
                for j in range(_F // 16):
                    zero_v[l, pl.ds(j * 16, 16)] = jnp.zeros((16,), jnp.float32)
            pltpu.sync_copy(zero_v, acc_sh)

        plsc.subcore_barrier()

        for i in range(nslots):
            b = i % _NBUF

            def body(i=i, b=b):
                if i >= 2:
                    mk_scat((i - 2) % _NBUF).wait()
                nxt = i + 2
                if nxt < nslots - 1:
                    start_loads(nxt, nxt % _NBUF)
                elif nxt == nslots - 1:
                    @pl.when(wid < last_cut)
                    def _():
                        start_loads(nxt, nxt % _NBUF)
                for d in mk_loads(i, b):
                    d.wait()
                mk_scat(b).start(add=True)

            if i < nslots - 1:
                body()
            else:
                pl.when(wid < last_cut)(body)

        @pl.when(wid < last_cut)
        def _():
            mk_scat((nslots - 2) % _NBUF).wait()
            mk_scat((nslots - 1) % _NBUF).wait()

        @pl.when(jnp.logical_not(wid < last_cut))
        def _():
            mk_scat((nslots - 3) % _NBUF).wait()
            mk_scat((nslots - 2) % _NBUF).wait()

        @pl.when(wid == nw - 1)
        def _():
            dtl.wait()
            dtx.wait()
            pltpu.sync_copy(xt_v, acc_sh.at[lblt_v], add=True)

        plsc.subcore_barrier()

        @pl.when(sid == 0)
        def _():
            pltpu.sync_copy(acc_sh, out_hbm.at[cid])

    return seg_kernel(x, labels)


def _seg_sum_tc(x, labels_tc):
    def body(lbl_ref, x_ref, o_ref):
        k = pl.program_id(0)

        @pl.when(k == 0)
        def _():
            o_ref[...] = jnp.zeros_like(o_ref)

        lbl = lbl_ref[...].reshape(1, _BTC)
        rows = lax.broadcasted_iota(jnp.int32, (_O, _BTC), 0)
        oh = (rows == jnp.broadcast_to(lbl, (_O, _BTC))).astype(jnp.bfloat16)
        xb = x_ref[...].astype(jnp.bfloat16)
        o_ref[...] += jnp.dot(oh, xb, preferred_element_type=jnp.float32)

    return pl.pallas_call(
        body,
        grid=(_KTC,),
        in_specs=[
            pl.BlockSpec((_BTC,), lambda k: (k,)),
            pl.BlockSpec((_BTC, _F), lambda k: (k, 0)),
        ],
        out_specs=pl.BlockSpec((_O, _F), lambda k: (0, 0)),
        out_shape=jax.ShapeDtypeStruct((_O, _F), jnp.float32),
    )(labels_tc, x)


def _combine_tc(partials, tc_part, w2, b):
    def tc_body(p_ref, t_ref, w_ref, b_ref, o_ref):
        seg = p_ref[0] + p_ref[1] + t_ref[pl.ds(0, _L), :]
        o_ref[...] = (
            jax.lax.dot(w_ref[...], seg, preferred_element_type=jnp.float32)
            + b_ref[0]
        )

    return pl.pallas_call(
        tc_body,
        out_shape=jax.ShapeDtypeStruct((_O, _F), jnp.float32),
    )(partials, tc_part, w2, b)


def kernel(x, node_labels, Param_W, Param_b):
    labels = node_labels.astype(jnp.int32)
    w2 = Param_W.reshape(_O, _L)
    sc_part = _seg_sum_sc(x, labels)
    tc_part = _seg_sum_tc(x, labels)
    out = _combine_tc(sc_part, tc_part, w2, Param_b)
    return out.reshape(_C, _O, _F)


# speedup vs baseline: 1.1006x; 1.0435x over previous
import functools

import jax
import jax.numpy as jnp
from jax import lax
from jax.experimental import pallas as pl
from jax.experimental.pallas import tpu as pltpu
from jax.experimental.pallas import tpu_sc as plsc

_C = 1
_O = 64
_L = 50
_N = 100000
_F = 128

_BTC = 4096
_KTC = 14
_NTC = _KTC * _BTC
_NSC = _N - _NTC

_LOAD = 128
_NLOAD = _NSC // _LOAD
_TROWS = _NSC - _NLOAD * _LOAD
_NBUF = 4


def _seg_sum_sc(x, labels):
    info = plsc.get_sparse_core_info()
    nc, ns = info.num_cores, info.num_subcores
    nw = nc * ns

    nslots = (_NLOAD + nw - 1) // nw
    last_cut = _NLOAD - (nslots - 1) * nw

    mesh = plsc.VectorSubcoreMesh(core_axis_name="c", subcore_axis_name="s")

    @functools.partial(
        pl.kernel,
        out_type=jax.ShapeDtypeStruct((nc, _L, _F), jnp.float32),
        mesh=mesh,
        scratch_types=[
            pltpu.VMEM((_NBUF, _LOAD, _F), jnp.float32),
            pltpu.VMEM((_NBUF, _LOAD), jnp.int32),
            pltpu.VMEM((_TROWS, _F), jnp.float32),
            pltpu.VMEM((_TROWS,), jnp.int32),
            pltpu.VMEM((_L, _F), jnp.float32),
            pltpu.VMEM_SHARED((_L, _F), jnp.float32),
            pltpu.SemaphoreType.DMA,
            pltpu.SemaphoreType.DMA,
            pltpu.SemaphoreType.DMA,
            pltpu.SemaphoreType.DMA,
            pltpu.SemaphoreType.DMA,
            pltpu.SemaphoreType.DMA,
            pltpu.SemaphoreType.DMA,
            pltpu.SemaphoreType.DMA,
            pltpu.SemaphoreType.DMA,
            pltpu.SemaphoreType.DMA,
            pltpu.SemaphoreType.DMA,
            pltpu.SemaphoreType.DMA,
            pltpu.SemaphoreType.DMA,
        ],
    )
    def seg_kernel(x_hbm, lbl_hbm, out_hbm, x_v, lbl_v, xt_v, lblt_v, zero_v,
                   acc_sh, sx0, sx1, sx2, sx3, sl0, sl1, sl2, sl3,
                   sc0, sc1, sc2, sc3, sq):
        cid = lax.axis_index("c")
        sid = lax.axis_index("s")
        wid = sid * nc + cid
        sx = (sx0, sx1, sx2, sx3)
        sl = (sl0, sl1, sl2, sl3)
        sc = (sc0, sc1, sc2, sc3)

        def mk_loads(i, b):
            m = wid + i * nw
            row0 = pl.multiple_of(_NTC + m * _LOAD, _LOAD)
            dl = pltpu.make_async_copy(
                lbl_hbm.at[pl.ds(row0, _LOAD)], lbl_v.at[b], sl[b])
            dx = pltpu.make_async_copy(
                x_hbm.at[pl.ds(row0, _LOAD), :], x_v.at[b], sx[b])
            return dl, dx

        def start_loads(i, b):
            for d in mk_loads(i, b):
                d.start()

        def mk_scat(b):
            return pltpu.make_async_copy(
                x_v.at[b], acc_sh.at[lbl_v.at[b]], sc[b])

        start_loads(0, 0)
        start_loads(1, 1)
        t0 = _NTC + _NLOAD * _LOAD
        dtl = pltpu.make_async_copy(lbl_hbm.at[pl.ds(t0, _TROWS)], lblt_v, sq)
        dtx = pltpu.make_async_copy(x_hbm.at[pl.ds(t0, _TROWS), :], xt_v, sq)

        @pl.when(wid == nw - 1)
        def _():
            dtl.start()
            dtx.start()

        @pl.when(sid == 0)
        def _():
            for l in range(_L):
                for j in range(_F // 16):
                    zero_v[l, pl.ds(j * 16, 16)] = jnp.zeros((16,), jnp.float32)
            pltpu.sync_copy(zero_v, acc_sh)

        plsc.subcore_barrier()

        for i in range(nslots):
            b = i % _NBUF

            def body(i=i, b=b):
                if i >= 2:
                    mk_scat((i - 2) % _NBUF).wait()
                nxt = i + 2
                if nxt < nslots - 1:
                    start_loads(nxt, nxt % _NBUF)
                elif nxt == nslots - 1:
                    @pl.when(wid < last_cut)
                    def _():
                        start_loads(nxt, nxt % _NBUF)
                for d in mk_loads(i, b):
                    d.wait()
                mk_scat(b).start(add=True)

            if i < nslots - 1:
                body()
            else:
                pl.when(wid < last_cut)(body)

        @pl.when(wid < last_cut)
        def _():
            mk_scat((nslots - 2) % _NBUF).wait()
            mk_scat((nslots - 1) % _NBUF).wait()

        @pl.when(jnp.logical_not(wid < last_cut))
        def _():
            mk_scat((nslots - 3) % _NBUF).wait()
            mk_scat((nslots - 2) % _NBUF).wait()

        @pl.when(wid == nw - 1)
        def _():
            dtl.wait()
            dtx.wait()
            pltpu.sync_copy(xt_v, acc_sh.at[lblt_v], add=True)

        plsc.subcore_barrier()

        @pl.when(sid == 0)
        def _():
            pltpu.sync_copy(acc_sh, out_hbm.at[cid])

    return seg_kernel(x, labels)


def _seg_sum_tc(x, labels_tc):
    def body(lbl_ref, x_ref, o_ref):
        k = pl.program_id(0)

        @pl.when(k == 0)
        def _():
            o_ref[...] = jnp.zeros_like(o_ref)

        lbl = lbl_ref[...].reshape(1, _BTC)
        rows = lax.broadcasted_iota(jnp.int32, (_O, _BTC), 0)
        oh = (rows == jnp.broadcast_to(lbl, (_O, _BTC))).astype(jnp.bfloat16)
        xb = x_ref[...].astype(jnp.bfloat16)
        o_ref[...] += jnp.dot(oh, xb, preferred_element_type=jnp.float32)

    return pl.pallas_call(
        body,
        grid=(_KTC,),
        in_specs=[
            pl.BlockSpec((_BTC,), lambda k: (k,)),
            pl.BlockSpec((_BTC, _F), lambda k: (k, 0)),
        ],
        out_specs=pl.BlockSpec((_O, _F), lambda k: (0, 0)),
        out_shape=jax.ShapeDtypeStruct((_O, _F), jnp.float32),
    )(labels_tc, x)


def _combine_tc(partials, tc_part, w2, b):
    def tc_body(p_ref, t_ref, w_ref, b_ref, o_ref):
        seg = p_ref[0] + p_ref[1] + t_ref[pl.ds(0, _L), :]
        o_ref[...] = (
            jax.lax.dot(w_ref[...], seg, preferred_element_type=jnp.float32)
            + b_ref[0]
        )

    return pl.pallas_call(
        tc_body,
        out_shape=jax.ShapeDtypeStruct((_O, _F), jnp.float32),
    )(partials, tc_part, w2, b)


def kernel(x, node_labels, Param_W, Param_b):
    labels = node_labels.astype(jnp.int32)
    w2 = Param_W.reshape(_O, _L)
    sc_part = _seg_sum_sc(x, labels)
    tc_part = _seg_sum_tc(x, labels)
    out = _combine_tc(sc_part, tc_part, w2, Param_b)
    return out.reshape(_C, _O, _F)


# speedup vs baseline: 1.1231x; 1.0204x over previous
import functools

import jax
import jax.numpy as jnp
from jax import lax
from jax.experimental import pallas as pl
from jax.experimental.pallas import tpu as pltpu
from jax.experimental.pallas import tpu_sc as plsc

_C = 1
_O = 64
_L = 50
_N = 100000
_F = 128

_BTC = 4096
_KTC = 15
_NTC = _KTC * _BTC
_NSC = _N - _NTC

_LOAD = 128
_NLOAD = _NSC // _LOAD
_TROWS = _NSC - _NLOAD * _LOAD
_NBUF = 4


def _seg_sum_sc(x, labels):
    info = plsc.get_sparse_core_info()
    nc, ns = info.num_cores, info.num_subcores
    nw = nc * ns

    nslots = (_NLOAD + nw - 1) // nw
    last_cut = _NLOAD - (nslots - 1) * nw

    mesh = plsc.VectorSubcoreMesh(core_axis_name="c", subcore_axis_name="s")

    @functools.partial(
        pl.kernel,
        out_type=jax.ShapeDtypeStruct((nc, _L, _F), jnp.float32),
        mesh=mesh,
        scratch_types=[
            pltpu.VMEM((_NBUF, _LOAD, _F), jnp.float32),
            pltpu.VMEM((_NBUF, _LOAD), jnp.int32),
            pltpu.VMEM((_TROWS, _F), jnp.float32),
            pltpu.VMEM((_TROWS,), jnp.int32),
            pltpu.VMEM((_L, _F), jnp.float32),
            pltpu.VMEM_SHARED((_L, _F), jnp.float32),
            pltpu.SemaphoreType.DMA,
            pltpu.SemaphoreType.DMA,
            pltpu.SemaphoreType.DMA,
            pltpu.SemaphoreType.DMA,
            pltpu.SemaphoreType.DMA,
            pltpu.SemaphoreType.DMA,
            pltpu.SemaphoreType.DMA,
            pltpu.SemaphoreType.DMA,
            pltpu.SemaphoreType.DMA,
            pltpu.SemaphoreType.DMA,
            pltpu.SemaphoreType.DMA,
            pltpu.SemaphoreType.DMA,
            pltpu.SemaphoreType.DMA,
        ],
    )
    def seg_kernel(x_hbm, lbl_hbm, out_hbm, x_v, lbl_v, xt_v, lblt_v, zero_v,
                   acc_sh, sx0, sx1, sx2, sx3, sl0, sl1, sl2, sl3,
                   sc0, sc1, sc2, sc3, sq):
        cid = lax.axis_index("c")
        sid = lax.axis_index("s")
        wid = sid * nc + cid
        sx = (sx0, sx1, sx2, sx3)
        sl = (sl0, sl1, sl2, sl3)
        sc = (sc0, sc1, sc2, sc3)

        def mk_loads(i, b):
            m = wid + i * nw
            row0 = pl.multiple_of(_NTC + m * _LOAD, _LOAD)
            dl = pltpu.make_async_copy(
                lbl_hbm.at[pl.ds(row0, _LOAD)], lbl_v.at[b], sl[b])
            dx = pltpu.make_async_copy(
                x_hbm.at[pl.ds(row0, _LOAD), :], x_v.at[b], sx[b])
            return dl, dx

        def start_loads(i, b):
            for d in mk_loads(i, b):
                d.start()

        def mk_scat(b):
            return pltpu.make_async_copy(
                x_v.at[b], acc_sh.at[lbl_v.at[b]], sc[b])

        start_loads(0, 0)
        start_loads(1, 1)
        t0 = _NTC + _NLOAD * _LOAD
        dtl = pltpu.make_async_copy(lbl_hbm.at[pl.ds(t0, _TROWS)], lblt_v, sq)
        dtx = pltpu.make_async_copy(x_hbm.at[pl.ds(t0, _TROWS), :], xt_v, sq)

        @pl.when(wid == nw - 1)
        def _():
            dtl.start()
            dtx.start()

        @pl.when(sid == 0)
        def _():
            for l in range(_L):
                for j in range(_F // 16):
                    zero_v[l, pl.ds(j * 16, 16)] = jnp.zeros((16,), jnp.float32)
            pltpu.sync_copy(zero_v, acc_sh)

        plsc.subcore_barrier()

        for i in range(nslots):
            b = i % _NBUF

            def body(i=i, b=b):
                if i >= 2:
                    mk_scat((i - 2) % _NBUF).wait()
                nxt = i + 2
                if nxt < nslots - 1:
                    start_loads(nxt, nxt % _NBUF)
                elif nxt == nslots - 1:
                    @pl.when(wid < last_cut)
                    def _():
                        start_loads(nxt, nxt % _NBUF)
                for d in mk_loads(i, b):
                    d.wait()
                mk_scat(b).start(add=True)

            if i < nslots - 1:
                body()
            else:
                pl.when(wid < last_cut)(body)

        @pl.when(wid < last_cut)
        def _():
            mk_scat((nslots - 2) % _NBUF).wait()
            mk_scat((nslots - 1) % _NBUF).wait()

        @pl.when(jnp.logical_not(wid < last_cut))
        def _():
            mk_scat((nslots - 3) % _NBUF).wait()
            mk_scat((nslots - 2) % _NBUF).wait()

        @pl.when(wid == nw - 1)
        def _():
            dtl.wait()
            dtx.wait()
            pltpu.sync_copy(xt_v, acc_sh.at[lblt_v], add=True)

        plsc.subcore_barrier()

        @pl.when(sid == 0)
        def _():
            pltpu.sync_copy(acc_sh, out_hbm.at[cid])

    return seg_kernel(x, labels)


def _seg_sum_tc(x, labels_tc):
    def body(lbl_ref, x_ref, o_ref):
        k = pl.program_id(0)

        @pl.when(k == 0)
        def _():
            o_ref[...] = jnp.zeros_like(o_ref)

        lbl = lbl_ref[...].reshape(1, _BTC)
        rows = lax.broadcasted_iota(jnp.int32, (_O, _BTC), 0)
        oh = (rows == jnp.broadcast_to(lbl, (_O, _BTC))).astype(jnp.bfloat16)
        xb = x_ref[...].astype(jnp.bfloat16)
        o_ref[...] += jnp.dot(oh, xb, preferred_element_type=jnp.float32)

    return pl.pallas_call(
        body,
        grid=(_KTC,),
        in_specs=[
            pl.BlockSpec((_BTC,), lambda k: (k,)),
            pl.BlockSpec((_BTC, _F), lambda k: (k, 0)),
        ],
        out_specs=pl.BlockSpec((_O, _F), lambda k: (0, 0)),
        out_shape=jax.ShapeDtypeStruct((_O, _F), jnp.float32),
    )(labels_tc, x)


def _combine_tc(partials, tc_part, w2, b):
    def tc_body(p_ref, t_ref, w_ref, b_ref, o_ref):
        seg = p_ref[0] + p_ref[1] + t_ref[pl.ds(0, _L), :]
        o_ref[...] = (
            jax.lax.dot(w_ref[...], seg, preferred_element_type=jnp.float32)
            + b_ref[0]
        )

    return pl.pallas_call(
        tc_body,
        out_shape=jax.ShapeDtypeStruct((_O, _F), jnp.float32),
    )(partials, tc_part, w2, b)


def kernel(x, node_labels, Param_W, Param_b):
    labels = node_labels.astype(jnp.int32)
    w2 = Param_W.reshape(_O, _L)
    sc_part = _seg_sum_sc(x, labels)
    tc_part = _seg_sum_tc(x, labels)
    out = _combine_tc(sc_part, tc_part, w2, Param_b)
    return out.reshape(_C, _O, _F)
